# baseline (speedup 1.0000x reference)
import functools
import jax
import jax.numpy as jnp
from jax import lax
from jax.experimental import pallas as pl
from jax.experimental.pallas import tpu as pltpu
from jax.experimental.pallas import tpu_sc as plsc

N = 10000
E = 320000
D = 128
NC = 2
NS = 16
NW = NC * NS
EPW = E // NW
CH = 80
NCH = EPW // CH
NP = 10112
RPT = NP // NS
RB = 2000


def _sc_care(h_tab, d_rep, s_rep, src3, dst3, zf, zc, ones16):
    mesh = plsc.VectorSubcoreMesh(core_axis_name="c", subcore_axis_name="s")

    @functools.partial(
        pl.kernel,
        mesh=mesh,
        out_type=[jax.ShapeDtypeStruct((NC, NP, D), jnp.float32),
                  jax.ShapeDtypeStruct((NC, NP, 16), jnp.float32)],
        compiler_params=pltpu.CompilerParams(use_tc_tiling_on_sc=False),
        scratch_types=[
            pltpu.VMEM((2, CH), jnp.int32),
            pltpu.VMEM((2, CH), jnp.int32),
            pltpu.VMEM((2, CH, D), jnp.float32),
            pltpu.VMEM((2, CH, 16), jnp.float32),
            pltpu.VMEM((2, CH, 16), jnp.float32),
            pltpu.VMEM((CH, D), jnp.float32),
            pltpu.VMEM((CH,), jnp.int32),
            pltpu.VMEM((CH, 16), jnp.float32),
            pltpu.VMEM_SHARED((NP, D), jnp.float32),
            pltpu.VMEM_SHARED((NP, 16), jnp.float32),
            pltpu.SemaphoreType.DMA,
            pltpu.SemaphoreType.DMA,
            pltpu.SemaphoreType.DMA,
            pltpu.SemaphoreType.DMA,
            pltpu.SemaphoreType.DMA,
            pltpu.SemaphoreType.DMA,
            pltpu.SemaphoreType.DMA,
            pltpu.SemaphoreType.DMA,
            pltpu.SemaphoreType.DMA,
            pltpu.SemaphoreType.DMA,
        ],
    )
    def k(h_hbm, d_hbm, s_hbm, src_hbm, dst_hbm, zf_hbm, zc_hbm, ones_hbm,
          outf_hbm, outc_hbm,
          src_v, dst_v, rows_v, drows_v, srows_v, sc_rows, sc_dst, ones_v,
          accf, accc,
          is0, is1, id0, id1, gs0, gs1, ds0, ds1, scf_sem, scc_sem):
        cid = lax.axis_index("c")
        sid = lax.axis_index("s")
        wid = sid * NC + cid
        isems = (is0, is1)
        idsems = (id0, id1)
        gsems = (gs0, gs1)
        dsems = (ds0, ds1)

        pltpu.sync_copy(ones_hbm, ones_v)
        pltpu.sync_copy(zf_hbm.at[pl.ds(sid * RPT, RPT)],
                        accf.at[pl.ds(sid * RPT, RPT)])
        pltpu.sync_copy(zc_hbm.at[pl.ds(sid * RPT, RPT)],
                        accc.at[pl.ds(sid * RPT, RPT)])
        plsc.subcore_barrier()

        def idx_dma(g, b):
            pltpu.async_copy(src_hbm.at[wid, g], src_v.at[b], isems[b])
            pltpu.async_copy(dst_hbm.at[wid, g], dst_v.at[b], idsems[b])

        def idx_wait(g, b):
            pltpu.make_async_copy(src_hbm.at[wid, g], src_v.at[b],
                                  isems[b]).wait()
            pltpu.make_async_copy(dst_hbm.at[wid, g], dst_v.at[b],
                                  idsems[b]).wait()

        def gather(b):
            pltpu.async_copy(h_hbm.at[src_v.at[b]], rows_v.at[b], gsems[b])
            pltpu.async_copy(d_hbm.at[dst_v.at[b]], drows_v.at[b], dsems[b])
            pltpu.async_copy(s_hbm.at[src_v.at[b]], srows_v.at[b], dsems[b])

        def gather_wait(b):
            pltpu.make_async_copy(h_hbm.at[src_v.at[b]], rows_v.at[b],
                                  gsems[b]).wait()
            pltpu.make_async_copy(d_hbm.at[dst_v.at[b]], drows_v.at[b],
                                  dsems[b]).wait()
            pltpu.make_async_copy(s_hbm.at[src_v.at[b]], srows_v.at[b],
                                  dsems[b]).wait()

        def scatter_wait():
            pltpu.make_async_copy(sc_rows, accf.at[sc_dst], scf_sem).wait()
            pltpu.make_async_copy(ones_v, accc.at[sc_dst], scc_sem).wait()

        def scale_and_scatter(b):
            gather_wait(b)

            def scale_body(r, c2):
                z = drows_v[b, r] + srows_v[b, r]
                av = 1.0 / (1.0 + jnp.exp(-z))
                for c in range(D // 16):
                    sc_rows[r, pl.ds(c * 16, 16)] = (
                        rows_v[b, r, pl.ds(c * 16, 16)] * av)
                return c2

            lax.fori_loop(0, CH, scale_body, 0)

            def dcopy_body(i, c2):
                sc_dst[pl.ds(i * 16, 16)] = dst_v[b, pl.ds(i * 16, 16)]
                return c2

            lax.fori_loop(0, CH // 16, dcopy_body, 0)
            pltpu.async_copy(sc_rows, accf.at[sc_dst], scf_sem, add=True)
            pltpu.async_copy(ones_v, accc.at[sc_dst], scc_sem, add=True)

        idx_dma(0, 0)
        idx_wait(0, 0)
        gather(0)
        idx_dma(1, 1)
        idx_wait(1, 1)
        gather(1)
        scale_and_scatter(0)
        idx_dma(2, 0)

        def chunk_step(g, b):
            qb = 1 - b
            idx_wait(g + 1, qb)
            gather(qb)
            scatter_wait()
            scale_and_scatter(b)
            idx_dma(g + 2, b)

        def pair_body(i, carry):
            chunk_step(2 * i + 1, 1)
            chunk_step(2 * i + 2, 0)
            return carry

        lax.fori_loop(0, (NCH - 3) // 2, pair_body, 0)
        chunk_step(NCH - 2, 1)
        scatter_wait()
        scale_and_scatter(0)
        scatter_wait()
        idx_wait(NCH, 1)
        plsc.subcore_barrier()

        pltpu.sync_copy(accf.at[pl.ds(sid * RPT, RPT)],
                        outf_hbm.at[cid, pl.ds(sid * RPT, RPT)])
        pltpu.sync_copy(accc.at[pl.ds(sid * RPT, RPT)],
                        outc_hbm.at[cid, pl.ds(sid * RPT, RPT)])

    return k(h_tab, d_rep, s_rep, src3, dst3, zf, zc, ones16)


def _enc_body(x_ref, w_ref, b_ref, ap_ref, ab_ref, h_ref, ds_ref):
    h = jnp.maximum(
        jnp.dot(x_ref[...], w_ref[...], preferred_element_type=jnp.float32)
        + b_ref[...], 0.0)
    h_ref[...] = h
    ds_ref[...] = (
        jnp.dot(h, ap_ref[...], preferred_element_type=jnp.float32)
        + ab_ref[...])


def _mid_body(f0_ref, f1_ref, c0_ref, c1_ref, ap_ref, ab_ref, h_ref, ds_ref):
    a = f0_ref[...] + f1_ref[...]
    cnt = jnp.maximum(c0_ref[...][:, :1] + c1_ref[...][:, :1], 1.0)
    h = jnp.maximum(a / cnt, 0.0)
    h_ref[...] = h
    ds_ref[...] = (
        jnp.dot(h, ap_ref[...], preferred_element_type=jnp.float32)
        + ab_ref[...])


def _cls_body(f0_ref, f1_ref, c0_ref, c1_ref, w_ref, b_ref, out_ref):
    a = f0_ref[...] + f1_ref[...]
    cnt = jnp.maximum(c0_ref[...][:, :1] + c1_ref[...][:, :1], 1.0)
    m = a / cnt
    out_ref[...] = (
        jnp.dot(m, w_ref[...], preferred_element_type=jnp.float32)
        + b_ref[...])


def _att_pack(att_W, att_b):
    ap = jnp.zeros((D, D), jnp.float32)
    ap = ap.at[:, 0].set(att_W[:D, 0]).at[:, 1].set(att_W[D:, 0])
    ab = jnp.zeros((1, D), jnp.float32).at[0, 0].set(att_b[0])
    return ap, ab


def kernel(x, edge_index, W_enc, b_enc, att1_W, att1_b, att2_W, att2_b,
           W_cls, b_cls):
    src = edge_index[0].reshape(NW, NCH, CH)
    dst = edge_index[1].reshape(NW, NCH, CH)
    src3 = jnp.concatenate([src, src[:, :2]], axis=1)
    dst3 = jnp.concatenate([dst, dst[:, :2]], axis=1)
    grid = (N // RB,)

    ap1, ab1 = _att_pack(att1_W, att1_b)
    ap2, ab2 = _att_pack(att2_W, att2_b)

    row_spec = pl.BlockSpec((RB, D), lambda i: (i, 0))
    cnt_spec = pl.BlockSpec((RB, 16), lambda i: (i, 0))
    w_spec = pl.BlockSpec((D, D), lambda i: (0, 0))
    b_spec = pl.BlockSpec((1, D), lambda i: (0, 0))

    h, ds1 = pl.pallas_call(
        _enc_body,
        grid=grid,
        in_specs=[row_spec, w_spec, b_spec, w_spec, b_spec],
        out_specs=[row_spec, row_spec],
        out_shape=[jax.ShapeDtypeStruct((N, D), jnp.float32),
                   jax.ShapeDtypeStruct((N, D), jnp.float32)],
    )(x, W_enc, b_enc.reshape(1, D), ap1, ab1)

    zf = jnp.zeros((NP, D), jnp.float32)
    zc = jnp.zeros((NP, 16), jnp.float32)
    ones16 = jnp.ones((CH, 16), jnp.float32)

    s1_rep = jnp.broadcast_to(ds1[:, 1:2], (N, 16))
    d1_rep = jnp.broadcast_to(ds1[:, 0:1], (N, 16))
    aggf1, aggc1 = _sc_care(h, d1_rep, s1_rep, src3, dst3, zf, zc, ones16)

    h2, ds2 = pl.pallas_call(
        _mid_body,
        grid=grid,
        in_specs=[row_spec, row_spec, cnt_spec, cnt_spec, w_spec, b_spec],
        out_specs=[row_spec, row_spec],
        out_shape=[jax.ShapeDtypeStruct((N, D), jnp.float32),
                   jax.ShapeDtypeStruct((N, D), jnp.float32)],
    )(aggf1[0, :N], aggf1[1, :N], aggc1[0, :N], aggc1[1, :N], ap2, ab2)

    s2_rep = jnp.broadcast_to(ds2[:, 1:2], (N, 16))
    d2_rep = jnp.broadcast_to(ds2[:, 0:1], (N, 16))
    aggf2, aggc2 = _sc_care(h2, d2_rep, s2_rep, src3, dst3, zf, zc, ones16)

    wc = jnp.zeros((D, D), jnp.float32).at[:, :2].set(W_cls)
    bc = jnp.zeros((1, D), jnp.float32).at[0, :2].set(b_cls)
    y = pl.pallas_call(
        _cls_body,
        grid=grid,
        in_specs=[row_spec, row_spec, cnt_spec, cnt_spec, w_spec, b_spec],
        out_specs=row_spec,
        out_shape=jax.ShapeDtypeStruct((N, D), jnp.float32),
    )(aggf2[0, :N], aggf2[1, :N], aggc2[0, :N], aggc2[1, :N], wc, bc)

    return y[:, :2]
